# baseline (speedup 1.0000x reference)
import jax
import jax.numpy as jnp
from jax import lax
from jax.experimental import pallas as pl
from jax.experimental.pallas import tpu as pltpu

_B = 4096
_S = 26
_C = 1000
_W = _S * _C
_RB = 16
_NBUF = 6


def _body(x_hbm, rep_ref, out_hbm, idx_scr, scr, insem, sems):
    i = pl.program_id(0)
    nb = pl.num_programs(0)
    slot = lax.rem(i, _NBUF)
    out2d = out_hbm

    @pl.when(i == 0)
    def _load_idx():
        pltpu.make_async_copy(x_hbm, idx_scr, insem).start()
        pltpu.make_async_copy(x_hbm, idx_scr, insem).wait()

    @pl.when(i >= _NBUF)
    def _wait_prev():
        pltpu.make_async_copy(
            scr.at[slot], out2d.at[pl.ds((i - _NBUF) * _RB, _RB)], sems.at[slot]
        ).wait()

    idx = idx_scr[pl.ds(i * _RB, _RB), :]
    iota3 = lax.broadcasted_iota(jnp.int32, (_RB, _S, _C), 2)
    oneh = (idx[:, :, None] == iota3).astype(jnp.float32)
    scr[slot] = oneh.reshape(_RB, _W)

    pltpu.make_async_copy(
        scr.at[slot], out2d.at[pl.ds(i * _RB, _RB)], sems.at[slot]
    ).start()

    @pl.when(i == nb - 1)
    def _drain():
        for k in range(_NBUF):
            j = i - (_NBUF - 1) + k
            pltpu.make_async_copy(
                scr.at[lax.rem(j, _NBUF)],
                out2d.at[pl.ds(j * _RB, _RB)],
                sems.at[lax.rem(j, _NBUF)],
            ).wait()


def kernel(x):
    xi = x.astype(jnp.int32)
    rep = jnp.repeat(jnp.eye(_S, dtype=jnp.float32), _C, axis=1)
    return pl.pallas_call(
        _body,
        grid=(_B // _RB,),
        in_specs=[
            pl.BlockSpec(memory_space=pl.ANY),
            pl.BlockSpec((_S, _W), lambda i: (0, 0)),
        ],
        out_specs=pl.BlockSpec(memory_space=pl.ANY),
        out_shape=jax.ShapeDtypeStruct((_B, _W), jnp.float32),
        scratch_shapes=[
            pltpu.VMEM((_B, _S), jnp.int32),
            pltpu.VMEM((_NBUF, _RB, _W), jnp.float32),
            pltpu.SemaphoreType.DMA,
            pltpu.SemaphoreType.DMA((_NBUF,)),
        ],
    )(xi, rep).reshape(_B, _S, _C)


# speedup vs baseline: 1.0401x; 1.0401x over previous
import jax
import jax.numpy as jnp
from jax import lax
from jax.experimental import pallas as pl
from jax.experimental.pallas import tpu as pltpu

_B = 4096
_S = 26
_C = 1000
_W = _S * _C
_RB = 64
_NBUF = 4


def _body(x_hbm, rep_ref, out_hbm, idx_scr, scr, insem, sems):
    i = pl.program_id(0)
    nb = pl.num_programs(0)
    slot = lax.rem(i, _NBUF)
    out2d = out_hbm

    @pl.when(i == 0)
    def _load_idx():
        pltpu.make_async_copy(x_hbm, idx_scr, insem).start()
        pltpu.make_async_copy(x_hbm, idx_scr, insem).wait()

    @pl.when(i >= _NBUF)
    def _wait_prev():
        pltpu.make_async_copy(
            scr.at[slot], out2d.at[pl.ds((i - _NBUF) * _RB, _RB)], sems.at[slot]
        ).wait()

    idx = idx_scr[pl.ds(i * _RB, _RB), :]
    iota3 = lax.broadcasted_iota(jnp.int32, (_RB, _S, _C), 2)
    oneh = (idx[:, :, None] == iota3).astype(jnp.float32)
    scr[slot] = oneh.reshape(_RB, _W)

    pltpu.make_async_copy(
        scr.at[slot], out2d.at[pl.ds(i * _RB, _RB)], sems.at[slot]
    ).start()

    @pl.when(i == nb - 1)
    def _drain():
        for k in range(_NBUF):
            j = i - (_NBUF - 1) + k
            pltpu.make_async_copy(
                scr.at[lax.rem(j, _NBUF)],
                out2d.at[pl.ds(j * _RB, _RB)],
                sems.at[lax.rem(j, _NBUF)],
            ).wait()


def kernel(x):
    xi = x.astype(jnp.int32)
    rep = jnp.repeat(jnp.eye(_S, dtype=jnp.float32), _C, axis=1)
    return pl.pallas_call(
        _body,
        grid=(_B // _RB,),
        in_specs=[
            pl.BlockSpec(memory_space=pl.ANY),
            pl.BlockSpec((_S, _W), lambda i: (0, 0)),
        ],
        out_specs=pl.BlockSpec(memory_space=pl.ANY),
        out_shape=jax.ShapeDtypeStruct((_B, _W), jnp.float32),
        scratch_shapes=[
            pltpu.VMEM((_B, _S), jnp.int32),
            pltpu.VMEM((_NBUF, _RB, _W), jnp.float32),
            pltpu.SemaphoreType.DMA,
            pltpu.SemaphoreType.DMA((_NBUF,)),
        ],
    )(xi, rep).reshape(_B, _S, _C)
